# baseline (speedup 1.0000x reference)
import functools
import math

import jax
import jax.numpy as jnp
from jax import lax
from jax.experimental import pallas as pl
from jax.experimental.pallas import tpu as pltpu

L = 2
D = 256
H = 8
DH = D // H
FF = 2048
NQ = 300
NQP = 304
B = 2
S = 4096
TOPK = 100
_SCALE = 1.0 / math.sqrt(DH)


def _layernorm(x, g, b, eps=1e-5):
    m = jnp.mean(x, axis=-1, keepdims=True)
    v = jnp.mean((x - m) ** 2, axis=-1, keepdims=True)
    return (x - m) / jnp.sqrt(v + eps) * g + b


_LOG2E = math.log2(math.e)


def _softmax_e(x):
    m = jnp.max(x, axis=-1, keepdims=True)
    return jnp.exp2(x - m).astype(jnp.bfloat16)


def _topk_mask(pc_col, pc_row):
    sig = jax.nn.sigmoid
    cx_c = sig(pc_col[:, 0:1]); cy_c = sig(pc_col[:, 1:2])
    w_c = sig(pc_col[:, 2:3]); h_c = sig(pc_col[:, 3:4])
    x1_c = cx_c - 0.5 * w_c; y1_c = cy_c - 0.5 * h_c
    x2_c = cx_c + 0.5 * w_c; y2_c = cy_c + 0.5 * h_c
    cx_r = sig(pc_row[0:1, :]); cy_r = sig(pc_row[1:2, :])
    w_r = sig(pc_row[2:3, :]); h_r = sig(pc_row[3:4, :])
    x1_r = cx_r - 0.5 * w_r; y1_r = cy_r - 0.5 * h_r
    x2_r = cx_r + 0.5 * w_r; y2_r = cy_r + 0.5 * h_r
    a_c = (x2_c - x1_c) * (y2_c - y1_c)
    a_r = (x2_r - x1_r) * (y2_r - y1_r)
    iw = jnp.maximum(jnp.minimum(x2_c, x2_r) - jnp.maximum(x1_c, x1_r), 0.0)
    ih = jnp.maximum(jnp.minimum(y2_c, y2_r) - jnp.maximum(y1_c, y1_r), 0.0)
    inter = iw * ih
    union = a_c + a_r - inter
    iou = inter / union
    cw = jnp.maximum(jnp.maximum(x2_c, x2_r) - jnp.minimum(x1_c, x1_r), 0.0)
    ch = jnp.maximum(jnp.maximum(y2_c, y2_r) - jnp.minimum(y1_c, y1_r), 0.0)
    area = cw * ch
    score = 1.0 - (iou - (area - union) / area)

    col = lax.broadcasted_iota(jnp.int32, (NQP, NQP), 1)
    score = jnp.where(col < NQ, score, jnp.inf)
    return lax.bitcast_convert_type(score, jnp.int32)


def _radix_select(bits):
    R = bits.shape[0]
    def step(i, carry):
        pref, kk, bitval = carry
        hi_mask = ~((bitval << 1) - 1)
        cand = ((bits & hi_mask) == pref) & ((bits & bitval) == 0)
        cnt = jnp.sum(cand.astype(jnp.float32), axis=1, keepdims=True)
        take0 = cnt >= kk
        pref = jnp.where(take0, pref, pref | bitval)
        kk = jnp.where(take0, kk, kk - cnt)
        return pref, kk, bitval >> 1
    pref0 = jnp.zeros((R, 1), jnp.int32)
    kk0 = jnp.full((R, 1), float(TOPK), jnp.float32)
    pref, _, _ = lax.fori_loop(0, 30, step, (pref0, kk0, jnp.int32(1 << 29)))

    less = bits < pref
    eq = bits == pref
    n_less = jnp.sum(less.astype(jnp.float32), axis=1, keepdims=True)
    need = float(TOPK) - n_less
    upper = (lax.broadcasted_iota(jnp.int32, (NQP, NQP), 0)
             < lax.broadcasted_iota(jnp.int32, (NQP, NQP), 1))
    rank = jnp.dot(eq.astype(jnp.float32), upper.astype(jnp.float32),
                   preferred_element_type=jnp.float32)
    return less | (eq & (rank < need))


def _decoder_body(x0_ref, qp_ref, mem_ref, memp_ref, pcc_ref, pcr_ref,
                  sa_wq, sa_bq, sa_wk, sa_bk, sa_wv, sa_bv, sa_wo, sa_bo,
                  ca_wq, ca_bq, ca_wk, ca_bk, ca_wv, ca_bv, ca_wo, ca_bo,
                  f_w1, f_b1, f_w2, f_b2,
                  ln1_g, ln1_b, ln2_g, ln2_b, ln3_g, ln3_b,
                  nrm_g, nrm_b,
                  out_ref, dec_ref):
    bf = jnp.bfloat16

    def mm(a, b):
        return jnp.dot(a, b, preferred_element_type=jnp.float32)

    def mm_t(a, b):
        return lax.dot_general(a, b, (((1,), (1,)), ((), ())),
                               preferred_element_type=jnp.float32)

    ones_s = jnp.ones((S, 1), bf)
    ones_q = jnp.ones((NQP, 1), bf)

    if True:
        bi = 0
        sel = _radix_select(_topk_mask(pcc_ref[bi], pcr_ref[bi]))
        qp_b = qp_ref[bi]
        x = x0_ref[bi]
        for l in range(L):
            qin = (x + qp_b).astype(bf)
            xbf = x.astype(bf)
            q = ((mm(qin, sa_wq[l]) + sa_bq[l])
                 * (_SCALE * _LOG2E)).astype(bf)
            k = (mm(qin, sa_wk[l]) + sa_bk[l]).astype(bf)
            v = (mm(xbf, sa_wv[l]) + sa_bv[l]).astype(bf)
            heads = []
            for h in range(H):
                sl = slice(h * DH, (h + 1) * DH)
                s = mm_t(q[:, sl], k[:, sl])
                e = _softmax_e(jnp.where(sel, s, -1e9))
                oz = mm(e, jnp.concatenate([v[:, sl], ones_q], axis=1))
                heads.append(oz[:, :DH] / oz[:, DH:DH + 1])
            sa_out = mm(jnp.concatenate(heads, axis=1).astype(bf),
                        sa_wo[l]) + sa_bo[l]
            x = _layernorm(x + sa_out, ln1_g[l], ln1_b[l])
            qc = ((mm((x + qp_b).astype(bf), ca_wq[l]) + ca_bq[l])
                  * (_SCALE * _LOG2E)).astype(bf)
            kc = (mm(memp_ref[bi], ca_wk[l]) + ca_bk[l]).astype(bf)
            vc = (mm(mem_ref[bi], ca_wv[l]) + ca_bv[l]).astype(bf)
            heads = []
            for h in range(H):
                sl = slice(h * DH, (h + 1) * DH)
                s = mm_t(qc[:, sl], kc[:, sl])
                e = _softmax_e(s)
                oz = mm(e, jnp.concatenate([vc[:, sl], ones_s], axis=1))
                z = oz[:, DH:DH + 1]
                if l == L - 1:
                    a = e.astype(jnp.float32) * ((1.0 / H) / z)
                    if h == 0:
                        dec_ref[bi] = a
                    else:
                        dec_ref[bi] += a
                heads.append(oz[:, :DH] / z)
            ca_out = mm(jnp.concatenate(heads, axis=1).astype(bf),
                        ca_wo[l]) + ca_bo[l]
            x = _layernorm(x + ca_out, ln2_g[l], ln2_b[l])
            h1 = jnp.maximum(mm(x.astype(bf), f_w1[l]) + f_b1[l], 0.0)
            y = mm(h1.astype(bf), f_w2[l]) + f_b2[l]
            x = _layernorm(x + y, ln3_g[l], ln3_b[l])
        out_ref[bi] = _layernorm(x, nrm_g[:], nrm_b[:])


@jax.jit
def _run(tgt, memory, pos, query_pos, pos_centers,
         sa_Wq, sa_bq, sa_Wk, sa_bk, sa_Wv, sa_bv, sa_Wo, sa_bo,
         ca_Wq, ca_bq, ca_Wk, ca_bk, ca_Wv, ca_bv, ca_Wo, ca_bo,
         ffn_W1, ffn_b1, ffn_W2, ffn_b2,
         ln1_g, ln1_b, ln2_g, ln2_b, ln3_g, ln3_b, norm_g, norm_b):
    pad_q = ((0, 0), (0, NQP - NQ), (0, 0))
    x0 = jnp.pad(tgt.transpose(1, 0, 2), pad_q)
    qp = jnp.pad(query_pos.transpose(1, 0, 2), pad_q)
    mem = memory.transpose(1, 0, 2)
    memp = (mem + pos.transpose(1, 0, 2)).astype(jnp.bfloat16)
    mem = mem.astype(jnp.bfloat16)
    pct = pos_centers.transpose(1, 0, 2)
    pcc = jnp.pad(pct, ((0, 0), (0, NQP - NQ), (0, 4)))
    pcr = jnp.pad(pct.transpose(0, 2, 1), ((0, 0), (0, 4), (0, NQP - NQ)))

    b3 = lambda a: a.reshape(L, 1, -1)
    wb = lambda a: a.astype(jnp.bfloat16)
    bsb = lambda *shp: pl.BlockSpec(shp, lambda b: (b,) + (0,) * (len(shp) - 1))
    bsc = lambda *shp: pl.BlockSpec(shp, lambda b: (0,) * len(shp))
    w_sp = [bsc(L, D, D), bsc(L, 1, D)] * 8
    outs = pl.pallas_call(
        _decoder_body,
        grid=(B,),
        in_specs=[bsb(1, NQP, D), bsb(1, NQP, D), bsb(1, S, D),
                  bsb(1, S, D), bsb(1, NQP, 8), bsb(1, 8, NQP)]
                 + w_sp
                 + [bsc(L, D, FF), bsc(L, 1, FF), bsc(L, FF, D), bsc(L, 1, D)]
                 + [bsc(L, 1, D)] * 6 + [bsc(1, D), bsc(1, D)],
        out_specs=[bsb(1, NQP, D), bsb(1, NQP, S)],
        out_shape=[
            jax.ShapeDtypeStruct((B, NQP, D), jnp.float32),
            jax.ShapeDtypeStruct((B, NQP, S), jnp.float32),
        ],
        compiler_params=pltpu.CompilerParams(
            dimension_semantics=("arbitrary",),
            vmem_limit_bytes=120 * 1024 * 1024,
        ),
    )(x0, qp, mem, memp, pcc, pcr,
      wb(sa_Wq), b3(sa_bq), wb(sa_Wk), b3(sa_bk), wb(sa_Wv), b3(sa_bv),
      wb(sa_Wo), b3(sa_bo),
      wb(ca_Wq), b3(ca_bq), wb(ca_Wk), b3(ca_bk), wb(ca_Wv), b3(ca_bv),
      wb(ca_Wo), b3(ca_bo),
      wb(ffn_W1), b3(ffn_b1), wb(ffn_W2), b3(ffn_b2),
      b3(ln1_g), b3(ln1_b), b3(ln2_g), b3(ln2_b), b3(ln3_g), b3(ln3_b),
      norm_g.reshape(1, D), norm_b.reshape(1, D))
    out_p, dec_p = outs
    out = out_p[:, :NQ, :].transpose(1, 0, 2)
    return out, pos_centers, dec_p[:, :NQ, :]


def kernel(*args):
    return _run(*args)


# speedup vs baseline: 1.1939x; 1.1939x over previous
import functools
import math

import jax
import jax.numpy as jnp
from jax import lax
from jax.experimental import pallas as pl
from jax.experimental.pallas import tpu as pltpu

L = 2
D = 256
H = 8
DH = D // H
FF = 2048
NQ = 300
NQP = 304
B = 2
S = 4096
TOPK = 100
_SCALE = 1.0 / math.sqrt(DH)


def _layernorm(x, g, b, eps=1e-5):
    m = jnp.mean(x, axis=-1, keepdims=True)
    v = jnp.mean((x - m) ** 2, axis=-1, keepdims=True)
    return (x - m) / jnp.sqrt(v + eps) * g + b


_LOG2E = math.log2(math.e)


def _softmax_e(x):
    m = jnp.max(x, axis=-1, keepdims=True)
    return jnp.exp2(x - m).astype(jnp.bfloat16)


def _topk_mask(pc_col, pc_row):
    sig = jax.nn.sigmoid
    cx_c = sig(pc_col[:, 0:1]); cy_c = sig(pc_col[:, 1:2])
    w_c = sig(pc_col[:, 2:3]); h_c = sig(pc_col[:, 3:4])
    x1_c = cx_c - 0.5 * w_c; y1_c = cy_c - 0.5 * h_c
    x2_c = cx_c + 0.5 * w_c; y2_c = cy_c + 0.5 * h_c
    cx_r = sig(pc_row[0:1, :]); cy_r = sig(pc_row[1:2, :])
    w_r = sig(pc_row[2:3, :]); h_r = sig(pc_row[3:4, :])
    x1_r = cx_r - 0.5 * w_r; y1_r = cy_r - 0.5 * h_r
    x2_r = cx_r + 0.5 * w_r; y2_r = cy_r + 0.5 * h_r
    a_c = (x2_c - x1_c) * (y2_c - y1_c)
    a_r = (x2_r - x1_r) * (y2_r - y1_r)
    iw = jnp.maximum(jnp.minimum(x2_c, x2_r) - jnp.maximum(x1_c, x1_r), 0.0)
    ih = jnp.maximum(jnp.minimum(y2_c, y2_r) - jnp.maximum(y1_c, y1_r), 0.0)
    inter = iw * ih
    union = a_c + a_r - inter
    iou = inter / union
    cw = jnp.maximum(jnp.maximum(x2_c, x2_r) - jnp.minimum(x1_c, x1_r), 0.0)
    ch = jnp.maximum(jnp.maximum(y2_c, y2_r) - jnp.minimum(y1_c, y1_r), 0.0)
    area = cw * ch
    score = 1.0 - (iou - (area - union) / area)

    col = lax.broadcasted_iota(jnp.int32, (NQP, NQP), 1)
    score = jnp.where(col < NQ, score, jnp.inf)
    return lax.bitcast_convert_type(score, jnp.int32)


def _radix_select(bits):
    R = bits.shape[0]
    def step(i, carry):
        pref, kk, bitval = carry
        hi_mask = ~((bitval << 1) - 1)
        cand = ((bits & hi_mask) == pref) & ((bits & bitval) == 0)
        cnt = jnp.sum(cand.astype(jnp.float32), axis=1, keepdims=True)
        take0 = cnt >= kk
        pref = jnp.where(take0, pref, pref | bitval)
        kk = jnp.where(take0, kk, kk - cnt)
        return pref, kk, bitval >> 1
    pref0 = jnp.zeros((R, 1), jnp.int32)
    kk0 = jnp.full((R, 1), float(TOPK), jnp.float32)
    pref, _, _ = lax.fori_loop(0, 30, step, (pref0, kk0, jnp.int32(1 << 29)))

    less = bits < pref
    eq = bits == pref
    n_less = jnp.sum(less.astype(jnp.float32), axis=1, keepdims=True)
    need = float(TOPK) - n_less
    upper = (lax.broadcasted_iota(jnp.int32, (NQP, NQP), 0)
             < lax.broadcasted_iota(jnp.int32, (NQP, NQP), 1))
    rank = jnp.dot(eq.astype(jnp.float32), upper.astype(jnp.float32),
                   preferred_element_type=jnp.float32)
    return less | (eq & (rank < need))


def _decoder_body(x0_ref, qp_ref, mem_ref, memp_ref, pcc_ref, pcr_ref,
                  sa_wq, sa_bq, sa_wk, sa_bk, sa_wv, sa_bv, sa_wo, sa_bo,
                  ca_wq, ca_bq, ca_wk, ca_bk, ca_wv, ca_bv, ca_wo, ca_bo,
                  f_w1, f_b1, f_w2, f_b2,
                  ln1_g, ln1_b, ln2_g, ln2_b, ln3_g, ln3_b,
                  nrm_g, nrm_b,
                  out_ref, dec_ref):
    bf = jnp.bfloat16

    def mm(a, b):
        return jnp.dot(a, b, preferred_element_type=jnp.float32)

    def mm_t(a, b):
        return lax.dot_general(a, b, (((1,), (1,)), ((), ())),
                               preferred_element_type=jnp.float32)

    ones_s = jnp.ones((S, 1), bf)
    ones_q = jnp.ones((NQP, 1), bf)

    for bi in range(B):
        sel = _radix_select(_topk_mask(pcc_ref[bi], pcr_ref[bi]))
        qp_b = qp_ref[bi]
        x = x0_ref[bi]
        for l in range(L):
            qin = (x + qp_b).astype(bf)
            xbf = x.astype(bf)
            q = ((mm(qin, sa_wq[l]) + sa_bq[l])
                 * (_SCALE * _LOG2E)).astype(bf)
            k = (mm(qin, sa_wk[l]) + sa_bk[l]).astype(bf)
            v = (mm(xbf, sa_wv[l]) + sa_bv[l]).astype(bf)
            heads = []
            for h in range(H):
                sl = slice(h * DH, (h + 1) * DH)
                s = mm_t(q[:, sl], k[:, sl])
                e = _softmax_e(jnp.where(sel, s, -1e9))
                oz = mm(e, jnp.concatenate([v[:, sl], ones_q], axis=1))
                heads.append(oz[:, :DH] / oz[:, DH:DH + 1])
            sa_out = mm(jnp.concatenate(heads, axis=1).astype(bf),
                        sa_wo[l]) + sa_bo[l]
            x = _layernorm(x + sa_out, ln1_g[l], ln1_b[l])
            qc = ((mm((x + qp_b).astype(bf), ca_wq[l]) + ca_bq[l])
                  * (_SCALE * _LOG2E)).astype(bf)
            kc = (mm(memp_ref[bi], ca_wk[l]) + ca_bk[l]).astype(bf)
            vc = (mm(mem_ref[bi], ca_wv[l]) + ca_bv[l]).astype(bf)
            heads = []
            for h in range(H):
                sl = slice(h * DH, (h + 1) * DH)
                s = mm_t(qc[:, sl], kc[:, sl])
                e = _softmax_e(s)
                oz = mm(e, jnp.concatenate([vc[:, sl], ones_s], axis=1))
                z = oz[:, DH:DH + 1]
                if l == L - 1:
                    a = e.astype(jnp.float32) * ((1.0 / H) / z)
                    if h == 0:
                        dec_ref[bi] = a
                    else:
                        dec_ref[bi] += a
                heads.append(oz[:, :DH] / z)
            ca_out = mm(jnp.concatenate(heads, axis=1).astype(bf),
                        ca_wo[l]) + ca_bo[l]
            x = _layernorm(x + ca_out, ln2_g[l], ln2_b[l])
            h1 = jnp.maximum(mm(x.astype(bf), f_w1[l]) + f_b1[l], 0.0)
            y = mm(h1.astype(bf), f_w2[l]) + f_b2[l]
            x = _layernorm(x + y, ln3_g[l], ln3_b[l])
        out_ref[bi] = _layernorm(x, nrm_g[:], nrm_b[:])


@jax.jit
def _run(tgt, memory, pos, query_pos, pos_centers,
         sa_Wq, sa_bq, sa_Wk, sa_bk, sa_Wv, sa_bv, sa_Wo, sa_bo,
         ca_Wq, ca_bq, ca_Wk, ca_bk, ca_Wv, ca_bv, ca_Wo, ca_bo,
         ffn_W1, ffn_b1, ffn_W2, ffn_b2,
         ln1_g, ln1_b, ln2_g, ln2_b, ln3_g, ln3_b, norm_g, norm_b):
    pad_q = ((0, 0), (0, NQP - NQ), (0, 0))
    x0 = jnp.pad(tgt.transpose(1, 0, 2), pad_q)
    qp = jnp.pad(query_pos.transpose(1, 0, 2), pad_q)
    mem = memory.transpose(1, 0, 2)
    memp = (mem + pos.transpose(1, 0, 2)).astype(jnp.bfloat16)
    mem = mem.astype(jnp.bfloat16)
    pct = pos_centers.transpose(1, 0, 2)
    pcc = jnp.pad(pct, ((0, 0), (0, NQP - NQ), (0, 4)))
    pcr = jnp.pad(pct.transpose(0, 2, 1), ((0, 0), (0, 4), (0, NQP - NQ)))

    b3 = lambda a: a.reshape(L, 1, -1)
    wb = lambda a: a.astype(jnp.bfloat16)
    outs = pl.pallas_call(
        _decoder_body,
        out_shape=[
            jax.ShapeDtypeStruct((B, NQP, D), jnp.float32),
            jax.ShapeDtypeStruct((B, NQP, S), jnp.float32),
        ],
        compiler_params=pltpu.CompilerParams(
            vmem_limit_bytes=120 * 1024 * 1024,
        ),
    )(x0, qp, mem, memp, pcc, pcr,
      wb(sa_Wq), b3(sa_bq), wb(sa_Wk), b3(sa_bk), wb(sa_Wv), b3(sa_bv),
      wb(sa_Wo), b3(sa_bo),
      wb(ca_Wq), b3(ca_bq), wb(ca_Wk), b3(ca_bk), wb(ca_Wv), b3(ca_bv),
      wb(ca_Wo), b3(ca_bo),
      wb(ffn_W1), b3(ffn_b1), wb(ffn_W2), b3(ffn_b2),
      b3(ln1_g), b3(ln1_b), b3(ln2_g), b3(ln2_b), b3(ln3_g), b3(ln3_b),
      norm_g.reshape(1, D), norm_b.reshape(1, D))
    out_p, dec_p = outs
    out = out_p[:, :NQ, :].transpose(1, 0, 2)
    return out, pos_centers, dec_p[:, :NQ, :]


def kernel(*args):
    return _run(*args)
